# baseline (speedup 1.0000x reference)
import functools

import jax
import jax.numpy as jnp
from jax import lax
from jax.experimental import pallas as pl

_NUM_CLASSES = 3
_NUM_ANCHORS = 3
_IMG_SIZE = 608
_G = 76
_S = _G * _G
_NCH = 7 + _NUM_CLASSES
_STRIDE = _IMG_SIZE / _G
_ANCHOR_W = (11.0, 23.0, 37.0)
_ANCHOR_H = (14.0, 27.0, 58.0)


def _decode_kernel(x_ref, o_ref):
    a = pl.program_id(1)
    p = x_ref[0, 0]

    sig = jax.nn.sigmoid(p)
    expv = jnp.minimum(jnp.exp(p), 1000.0)

    c = lax.broadcasted_iota(jnp.int32, (_NCH, _S), 0)
    s = lax.broadcasted_iota(jnp.int32, (_NCH, _S), 1)
    gx = (s % _G).astype(jnp.float32)
    gy = (s // _G).astype(jnp.float32)

    aw = jnp.where(a == 0, _ANCHOR_W[0], jnp.where(a == 1, _ANCHOR_W[1], _ANCHOR_W[2]))
    ah = jnp.where(a == 0, _ANCHOR_H[0], jnp.where(a == 1, _ANCHOR_H[1], _ANCHOR_H[2]))
    aw = aw.astype(jnp.float32)
    ah = ah.astype(jnp.float32)

    val = jnp.where(
        c <= 1,
        (sig + jnp.where(c == 0, gx, gy)) * _STRIDE,
        jnp.where(
            c <= 3,
            expv * jnp.where(c == 2, aw, ah),
            jnp.where(c <= 5, p, sig),
        ),
    )
    o_ref[0, 0] = val.T


@jax.jit
def kernel(x):
    B = x.shape[0]
    xr = x.reshape(B, _NUM_ANCHORS, _NCH, _S)
    out = pl.pallas_call(
        _decode_kernel,
        grid=(B, _NUM_ANCHORS),
        in_specs=[
            pl.BlockSpec((1, 1, _NCH, _S), lambda b, a: (b, a, 0, 0)),
        ],
        out_specs=pl.BlockSpec((1, 1, _S, _NCH), lambda b, a: (b, a, 0, 0)),
        out_shape=jax.ShapeDtypeStruct((B, _NUM_ANCHORS, _S, _NCH), jnp.float32),
    )(xr)
    return out.reshape(B, _NUM_ANCHORS * _S, _NCH)
